# Initial kernel scaffold; baseline (speedup 1.0000x reference)
import jax
import jax.numpy as jnp
from jax.experimental import pallas as pl


def kernel(x, edge_index, W1, b1, W2, b2):
    raise NotImplementedError("write your pallas kernel here")


# speedup vs baseline: 12.3319x; 12.3319x over previous
import functools
import jax
import jax.numpy as jnp
from jax import lax
from jax.experimental import pallas as pl
from jax.experimental.pallas import tpu as pltpu
from jax.experimental.pallas import tpu_sc as plsc

N = 10000
E = 320000
D_IN = 128
D_HID = 128
N_CLS = 40
D2P = 128

NC = 2
NS = 16
NW = NC * NS
E_PER_W = E // NW
CH = 80
NCHUNK = E_PER_W // CH
N_PAD = 10240
ROWS_PER_TILE = N_PAD // NS


def _sc_mesh():
    return plsc.VectorSubcoreMesh(core_axis_name="c", subcore_axis_name="s")


@functools.partial(
    pl.kernel,
    mesh=_sc_mesh(),
    out_type=jax.ShapeDtypeStruct((NC, N_PAD, 128), jnp.float32),
    scratch_types=[
        pltpu.VMEM((CH,), jnp.int32),
        pltpu.VMEM((CH, 128), jnp.float32),
        pltpu.VMEM_SHARED((N_PAD, 128), jnp.float32),
    ],
)
def _deg_pass(dst_hbm, ones_hbm, zeros_hbm, out_hbm, dst_v, ones_v, acc_sh):
    c = lax.axis_index("c")
    s = lax.axis_index("s")
    wid = c * NS + s
    base = wid * E_PER_W
    rbase = s * ROWS_PER_TILE

    pltpu.sync_copy(ones_hbm, ones_v)
    pltpu.sync_copy(zeros_hbm.at[pl.ds(rbase, ROWS_PER_TILE)],
                    acc_sh.at[pl.ds(rbase, ROWS_PER_TILE)])
    plsc.subcore_barrier()

    def body(j, carry):
        off = base + j * CH
        pltpu.sync_copy(dst_hbm.at[pl.ds(off, CH)], dst_v)
        pltpu.sync_copy(ones_v, acc_sh.at[dst_v], add=True)
        return carry

    lax.fori_loop(0, NCHUNK, body, 0)
    plsc.subcore_barrier()
    pltpu.sync_copy(acc_sh.at[pl.ds(rbase, ROWS_PER_TILE)],
                    out_hbm.at[c, pl.ds(rbase, ROWS_PER_TILE)])


def _make_agg(D):
    def body_fn(y_hbm, src_hbm, dst_hbm, zeros_hbm, out_hbm,
                src_v, dst_v, rows_v, acc_sh, sem):
        c = lax.axis_index("c")
        s = lax.axis_index("s")
        wid = c * NS + s
        base = wid * E_PER_W
        rbase = s * ROWS_PER_TILE

        pltpu.sync_copy(zeros_hbm.at[pl.ds(rbase, ROWS_PER_TILE)],
                        acc_sh.at[pl.ds(rbase, ROWS_PER_TILE)])
        plsc.subcore_barrier()

        def body(j, carry):
            off = base + j * CH
            pltpu.sync_copy(src_hbm.at[pl.ds(off, CH)], src_v)
            pltpu.sync_copy(dst_hbm.at[pl.ds(off, CH)], dst_v)
            pltpu.async_copy(y_hbm.at[src_v], rows_v, sem).wait()
            pltpu.sync_copy(rows_v, acc_sh.at[dst_v], add=True)
            return carry

        lax.fori_loop(0, NCHUNK, body, 0)
        plsc.subcore_barrier()
        pltpu.sync_copy(acc_sh.at[pl.ds(rbase, ROWS_PER_TILE)],
                        out_hbm.at[c, pl.ds(rbase, ROWS_PER_TILE)])

    return pl.kernel(
        body_fn,
        mesh=_sc_mesh(),
        out_type=jax.ShapeDtypeStruct((NC, N_PAD, D), jnp.float32),
        scratch_types=[
            pltpu.VMEM((CH,), jnp.int32),
            pltpu.VMEM((CH,), jnp.int32),
            pltpu.VMEM((CH, D), jnp.float32),
            pltpu.VMEM_SHARED((N_PAD, D), jnp.float32),
            pltpu.SemaphoreType.DMA,
        ],
    )


_agg128 = _make_agg(D_HID)
_agg48 = _make_agg(D2P)


_BLK = 1000


def _xw_body(x_ref, w_ref, o_ref):
    o_ref[...] = jnp.dot(x_ref[...], w_ref[...],
                         preferred_element_type=jnp.float32)


def _scale1_body(parts_ref, xw_ref, dinv_ref, y1_ref):
    deg = parts_ref[0][:, :1] + parts_ref[1][:, :1] + 1.0
    dinv = lax.rsqrt(jnp.clip(deg, 1.0, None))
    dinv_ref[...] = jnp.broadcast_to(dinv, (_BLK, 16))
    y1_ref[...] = xw_ref[...] * dinv


def _layer2_body(p_ref, y1_ref, dinv_ref, b1_ref, w2_ref, y2_ref):
    dinv = dinv_ref[:, :1]
    z = p_ref[0] + p_ref[1] + y1_ref[...]
    h = jnp.maximum(z * dinv + b1_ref[...], 0.0)
    y2_ref[...] = jnp.dot(h, w2_ref[...],
                          preferred_element_type=jnp.float32) * dinv


def _epilogue_body(p_ref, y2_ref, dinv_ref, b2_ref, o_ref):
    dinv = dinv_ref[:, :1]
    z = p_ref[0] + p_ref[1] + y2_ref[...]
    o_ref[...] = z * dinv + b2_ref[...]


def kernel(x, edge_index, W1, b1, W2, b2):
    f32 = jnp.float32
    src = edge_index[0]
    dst = edge_index[1]
    ones128 = jnp.ones((CH, 128), f32)

    zeros128 = jnp.zeros((N_PAD, D_HID), f32)
    zeros48 = jnp.zeros((N_PAD, D2P), f32)
    W2p = jnp.zeros((D_HID, D2P), f32).at[:, :N_CLS].set(W2)
    b1r = b1.reshape(1, D_HID)
    b2p = jnp.zeros((1, D2P), f32).at[0, :N_CLS].set(b2)

    grid = N // _BLK

    deg_parts = _deg_pass(dst, ones128, zeros128)

    xw1 = pl.pallas_call(
        _xw_body,
        grid=(grid,),
        in_specs=[pl.BlockSpec((_BLK, D_IN), lambda i: (i, 0)),
                  pl.BlockSpec((D_IN, D_HID), lambda i: (0, 0))],
        out_specs=pl.BlockSpec((_BLK, D_HID), lambda i: (i, 0)),
        out_shape=jax.ShapeDtypeStruct((N, D_HID), f32),
    )(x, W1)

    dinv16, y1 = pl.pallas_call(
        _scale1_body,
        grid=(grid,),
        in_specs=[pl.BlockSpec((NC, _BLK, 128), lambda i: (0, i, 0)),
                  pl.BlockSpec((_BLK, D_HID), lambda i: (i, 0))],
        out_specs=[pl.BlockSpec((_BLK, 16), lambda i: (i, 0)),
                   pl.BlockSpec((_BLK, D_HID), lambda i: (i, 0))],
        out_shape=[jax.ShapeDtypeStruct((N, 16), f32),
                   jax.ShapeDtypeStruct((N, D_HID), f32)],
    )(deg_parts, xw1)

    p1 = _agg128(y1, src, dst, zeros128)

    y2 = pl.pallas_call(
        _layer2_body,
        grid=(grid,),
        in_specs=[pl.BlockSpec((NC, _BLK, D_HID), lambda i: (0, i, 0)),
                  pl.BlockSpec((_BLK, D_HID), lambda i: (i, 0)),
                  pl.BlockSpec((_BLK, 16), lambda i: (i, 0)),
                  pl.BlockSpec((1, D_HID), lambda i: (0, 0)),
                  pl.BlockSpec((D_HID, D2P), lambda i: (0, 0))],
        out_specs=pl.BlockSpec((_BLK, D2P), lambda i: (i, 0)),
        out_shape=jax.ShapeDtypeStruct((N, D2P), f32),
    )(p1, y1, dinv16, b1r, W2p)

    p2 = _agg48(y2, src, dst, zeros48)

    outp = pl.pallas_call(
        _epilogue_body,
        grid=(grid,),
        in_specs=[pl.BlockSpec((NC, _BLK, D2P), lambda i: (0, i, 0)),
                  pl.BlockSpec((_BLK, D2P), lambda i: (i, 0)),
                  pl.BlockSpec((_BLK, 16), lambda i: (i, 0)),
                  pl.BlockSpec((1, D2P), lambda i: (0, 0))],
        out_specs=pl.BlockSpec((_BLK, D2P), lambda i: (i, 0)),
        out_shape=jax.ShapeDtypeStruct((N, D2P), f32),
    )(p2, y2, dinv16, b2p)

    return outp[:, :N_CLS]
